# baseline (speedup 1.0000x reference)
import functools

import jax
import jax.numpy as jnp
from jax import lax
from jax.experimental import pallas as pl
from jax.experimental.pallas import tpu as pltpu
from jax.experimental.pallas import tpu_sc as plsc

MAX_WEEKS = 160
EMBED_DIM = 64
BATCH = 16384
HIST = 200

N = BATCH * HIST
NC, NS = 2, 16
NW = NC * NS
PER_W = N // NW
CHUNK = 512
GROUPS = CHUNK // 16
N_ITER = PER_W // CHUNK
NBUF = 2
N_OUTER = N_ITER // NBUF
TABLE_WORDS = MAX_WEEKS * EMBED_DIM
CHUNK_WORDS = CHUNK * EMBED_DIM

_mesh = plsc.VectorSubcoreMesh(core_axis_name="c", subcore_axis_name="s")


@functools.partial(
    pl.kernel,
    out_type=jax.ShapeDtypeStruct((N * EMBED_DIM,), jnp.float32),
    mesh=_mesh,
    scratch_types=[
        pltpu.VMEM((NBUF, CHUNK), jnp.int32),
        pltpu.VMEM((NBUF, CHUNK_WORDS), jnp.float32),
        pltpu.VMEM((TABLE_WORDS,), jnp.float32),
        pltpu.SemaphoreType.DMA,
        pltpu.SemaphoreType.DMA,
        pltpu.SemaphoreType.DMA,
    ],
    compiler_params=pltpu.CompilerParams(
        use_tc_tiling_on_sc=False, needs_layout_passes=False
    ),
)
def _gather_kernel(idx_hbm, table_hbm, out_hbm, idx_v, rows_v, table_v,
                   isem, osem_a, osem_b):
    wid = lax.axis_index("s") * NC + lax.axis_index("c")

    pltpu.sync_copy(table_hbm, table_v)

    base_idx = wid * PER_W
    base_out = wid * PER_W * EMBED_DIM
    osems = [osem_a, osem_b]
    riota = lax.iota(jnp.int32, 16) * EMBED_DIM

    pltpu.async_copy(idx_hbm.at[pl.ds(base_idx, CHUNK)], idx_v.at[0], isem)

    def outer(o, carry):
        for b in range(NBUF):
            t = NBUF * o + b
            pltpu.make_async_copy(
                idx_hbm.at[pl.ds(0, CHUNK)], idx_v.at[b], isem
            ).wait()

            @pl.when(t + 1 < N_ITER)
            def _prefetch():
                pltpu.async_copy(
                    idx_hbm.at[pl.ds(base_idx + (t + 1) * CHUNK, CHUNK)],
                    idx_v.at[1 - b],
                    isem,
                )

            @pl.when(t >= NBUF)
            def _drain_prev_out():
                pltpu.make_async_copy(
                    rows_v.at[b], out_hbm.at[pl.ds(0, CHUNK_WORDS)], osems[b]
                ).wait()

            rows_b = rows_v.at[b]
            idx_b = idx_v.at[b]

            def group(g, carry2):
                idx16 = idx_b[pl.ds(g * 16, 16)]
                src = idx16 * EMBED_DIM
                dst = riota + g * (16 * EMBED_DIM)
                for d in range(EMBED_DIM):
                    vals = plsc.load_gather(table_v, [src + d])
                    plsc.store_scatter(rows_b, [dst + d], vals)
                return carry2

            lax.fori_loop(0, GROUPS, group, 0)

            pltpu.async_copy(
                rows_b,
                out_hbm.at[pl.ds(base_out + t * CHUNK_WORDS, CHUNK_WORDS)],
                osems[b],
            )
        return carry

    lax.fori_loop(0, N_OUTER, outer, 0)

    for b in range(NBUF):
        pltpu.make_async_copy(
            rows_v.at[b], out_hbm.at[pl.ds(0, CHUNK_WORDS)], osems[b]
        ).wait()


def kernel(week_numbers, week_embed):
    idx = week_numbers.reshape(N).astype(jnp.int32)
    out = _gather_kernel(idx, week_embed.reshape(TABLE_WORDS))
    return out.reshape(BATCH, HIST, EMBED_DIM)


# speedup vs baseline: 1.1415x; 1.1415x over previous
import functools

import jax
import jax.numpy as jnp
from jax import lax
from jax.experimental import pallas as pl
from jax.experimental.pallas import tpu as pltpu
from jax.experimental.pallas import tpu_sc as plsc

MAX_WEEKS = 160
EMBED_DIM = 64
BATCH = 16384
HIST = 200

N = BATCH * HIST
NC, NS = 2, 16
NW = NC * NS
PER_W = N // NW
CHUNK = 512
GROUPS = CHUNK // 16
N_ITER = PER_W // CHUNK
NBUF = 2
N_OUTER = N_ITER // NBUF
TABLE_WORDS = MAX_WEEKS * EMBED_DIM
CHUNK_WORDS = CHUNK * EMBED_DIM

_mesh = plsc.VectorSubcoreMesh(core_axis_name="c", subcore_axis_name="s")


@functools.partial(
    pl.kernel,
    out_type=jax.ShapeDtypeStruct((N * EMBED_DIM,), jnp.float32),
    mesh=_mesh,
    scratch_types=[
        pltpu.VMEM((NBUF, CHUNK), jnp.int32),
        pltpu.VMEM((NBUF, CHUNK_WORDS), jnp.float32),
        pltpu.VMEM((TABLE_WORDS,), jnp.float32),
        pltpu.SemaphoreType.DMA,
        pltpu.SemaphoreType.DMA,
        pltpu.SemaphoreType.DMA,
    ],
    compiler_params=pltpu.CompilerParams(
        use_tc_tiling_on_sc=False, needs_layout_passes=False
    ),
)
def _gather_kernel(idx_hbm, table_hbm, out_hbm, idx_v, rows_v, table_v,
                   isem, osem_a, osem_b):
    wid = lax.axis_index("s") * NC + lax.axis_index("c")

    pltpu.sync_copy(table_hbm, table_v)

    base_idx = wid * PER_W
    base_out = wid * PER_W * EMBED_DIM
    osems = [osem_a, osem_b]
    riota = lax.iota(jnp.int32, 16) * EMBED_DIM

    pltpu.async_copy(idx_hbm.at[pl.ds(base_idx, CHUNK)], idx_v.at[0], isem)

    def outer(o, carry):
        for b in range(NBUF):
            t = NBUF * o + b
            pltpu.make_async_copy(
                idx_hbm.at[pl.ds(0, CHUNK)], idx_v.at[b], isem
            ).wait()

            @pl.when(t + 1 < N_ITER)
            def _prefetch():
                pltpu.async_copy(
                    idx_hbm.at[pl.ds(base_idx + (t + 1) * CHUNK, CHUNK)],
                    idx_v.at[1 - b],
                    isem,
                )

            @pl.when(t >= NBUF)
            def _drain_prev_out():
                pltpu.make_async_copy(
                    rows_v.at[b], out_hbm.at[pl.ds(0, CHUNK_WORDS)], osems[b]
                ).wait()

            rows_b = rows_v.at[b]
            idx_b = idx_v.at[b]

            @plsc.parallel_loop(0, GROUPS, step=1, unroll=2)
            def group(g):
                idx16 = idx_b[pl.ds(g * 16, 16)]
                src = idx16 * EMBED_DIM
                dst = riota + g * (16 * EMBED_DIM)
                for d in range(EMBED_DIM):
                    vals = plsc.load_gather(table_v, [src + d])
                    plsc.store_scatter(rows_b, [dst + d], vals)

            pltpu.async_copy(
                rows_b,
                out_hbm.at[pl.ds(base_out + t * CHUNK_WORDS, CHUNK_WORDS)],
                osems[b],
            )
        return carry

    lax.fori_loop(0, N_OUTER, outer, 0)

    for b in range(NBUF):
        pltpu.make_async_copy(
            rows_v.at[b], out_hbm.at[pl.ds(0, CHUNK_WORDS)], osems[b]
        ).wait()


def kernel(week_numbers, week_embed):
    idx = week_numbers.reshape(N).astype(jnp.int32)
    out = _gather_kernel(idx, week_embed.reshape(TABLE_WORDS))
    return out.reshape(BATCH, HIST, EMBED_DIM)


# speedup vs baseline: 4.0962x; 3.5885x over previous
import functools

import jax
import jax.numpy as jnp
from jax import lax
from jax.experimental import pallas as pl
from jax.experimental.pallas import tpu as pltpu
from jax.experimental.pallas import tpu_sc as plsc

MAX_WEEKS = 160
EMBED_DIM = 64
BATCH = 16384
HIST = 200

N = BATCH * HIST
NC, NS = 2, 16
NW = NC * NS
PER_W = N // NW
IDX_MINOR = 128
CHUNK = 512
ROWS_PER = CHUNK // IDX_MINOR
N_SPMEM = 3
N_ITER = PER_W // CHUNK
NBUF = 2
N_OUTER = N_ITER // NBUF

_mesh = plsc.VectorSubcoreMesh(core_axis_name="c", subcore_axis_name="s")


@functools.partial(
    pl.kernel,
    out_type=jax.ShapeDtypeStruct((N, EMBED_DIM), jnp.float32),
    mesh=_mesh,
    scratch_types=[
        pltpu.VMEM((NBUF, ROWS_PER, IDX_MINOR), jnp.int32),
        pltpu.VMEM((NBUF, CHUNK, EMBED_DIM), jnp.float32),
        pltpu.VMEM_SHARED((MAX_WEEKS, EMBED_DIM), jnp.float32),
        pltpu.SemaphoreType.DMA,
        pltpu.SemaphoreType.DMA,
        pltpu.SemaphoreType.DMA,
        pltpu.SemaphoreType.DMA,
    ],
    compiler_params=pltpu.CompilerParams(use_tc_tiling_on_sc=False),
)
def _gather_kernel(idx_hbm, table_hbm, out_hbm, idx_v, rows_v, table_v,
                   isem, gsem, osem_a, osem_b):
    wid = lax.axis_index("s") * NC + lax.axis_index("c")

    @pl.when(lax.axis_index("s") == 0)
    def _stage_table():
        pltpu.sync_copy(table_hbm, table_v)

    plsc.subcore_barrier()

    base_irow = wid * (PER_W // IDX_MINOR)
    base_out = wid * PER_W
    osems = [osem_a, osem_b]

    pltpu.async_copy(idx_hbm.at[pl.ds(base_irow, ROWS_PER)], idx_v.at[0], isem)

    def outer(o, carry):
        for b in range(NBUF):
            t = NBUF * o + b
            pltpu.make_async_copy(
                idx_hbm.at[pl.ds(0, ROWS_PER)], idx_v.at[b], isem
            ).wait()

            @pl.when(t + 1 < N_ITER)
            def _prefetch():
                irow = base_irow + (t + 1) * ROWS_PER
                pltpu.async_copy(
                    idx_hbm.at[pl.ds(irow, ROWS_PER)], idx_v.at[1 - b], isem
                )

            @pl.when(t >= NBUF)
            def _drain_prev_out():
                pltpu.make_async_copy(
                    rows_v.at[b], out_hbm.at[pl.ds(0, CHUNK)], osems[b]
                ).wait()

            handles = [
                pltpu.async_copy(
                    table_v.at[idx_v.at[b].at[j]],
                    rows_v.at[b].at[pl.ds(j * IDX_MINOR, IDX_MINOR)],
                    gsem,
                )
                for j in range(ROWS_PER)
            ]
            for h in handles:
                h.wait()

            pltpu.async_copy(
                rows_v.at[b],
                out_hbm.at[pl.ds(base_out + t * CHUNK, CHUNK)],
                osems[b],
            )
        return carry

    lax.fori_loop(0, N_OUTER, outer, 0)

    for b in range(NBUF):
        pltpu.make_async_copy(
            rows_v.at[b], out_hbm.at[pl.ds(0, CHUNK)], osems[b]
        ).wait()


def kernel(week_numbers, week_embed):
    idx = week_numbers.reshape(N).astype(jnp.int32).reshape(N // IDX_MINOR, IDX_MINOR)
    out = _gather_kernel(idx, week_embed)
    return out.reshape(BATCH, HIST, EMBED_DIM)
